# Initial kernel scaffold; baseline (speedup 1.0000x reference)
import jax
import jax.numpy as jnp
from jax.experimental import pallas as pl


def kernel(features, coords, W_ft, b_ft, W_coord, b_coord, W_feat, b_feat, gamma, beta):
    raise NotImplementedError("write your pallas kernel here")


# speedup vs baseline: 26.7140x; 26.7140x over previous
import functools
from math import sqrt

import jax
import jax.numpy as jnp
from jax import lax
from jax.experimental import pallas as pl
from jax.experimental.pallas import tpu as pltpu
from jax.experimental.pallas import tpu_sc as plsc

_N = 10000
_C = 128
_B = 4
_K = 16
_NPB = _N // _B
_NP = 2560
_RT = 256
_CHUNK = 8
_NCHUNK = _N // _CHUNK
_NW = 32
_GMAX = -(-_NCHUNK // _NW)


def _pre_body(f_ref, c_ref, wfeat_ref, wcoord_ref, wft_ref, bft_ref, out_ref):
    f = f_ref[...]
    xyz = c_ref[:, 1:4]
    out_ref[:, 0:_C] = jnp.dot(f, wfeat_ref[...], preferred_element_type=jnp.float32)
    out_ref[:, _C:2 * _C] = jnp.dot(xyz, wcoord_ref[...], preferred_element_type=jnp.float32)
    out_ref[:, 2 * _C:3 * _C] = (
        jnp.dot(f, wft_ref[...], preferred_element_type=jnp.float32) + bft_ref[...]
    )


def _precompute_table(features, coords, W_feat, W_coord, W_ft, b_ft):
    rows = 1000
    grid = _N // rows
    return pl.pallas_call(
        _pre_body,
        grid=(grid,),
        in_specs=[
            pl.BlockSpec((rows, _C), lambda i: (i, 0)),
            pl.BlockSpec((rows, 4), lambda i: (i, 0)),
            pl.BlockSpec((_C, _C), lambda i: (0, 0)),
            pl.BlockSpec((3, _C), lambda i: (0, 0)),
            pl.BlockSpec((_C, _C), lambda i: (0, 0)),
            pl.BlockSpec((1, _C), lambda i: (0, 0)),
        ],
        out_specs=pl.BlockSpec((rows, 3 * _C), lambda i: (i, 0)),
        out_shape=jax.ShapeDtypeStruct((_N, 3 * _C), jnp.float32),
    )(features, coords, W_feat, W_coord, W_ft, b_ft.reshape(1, _C))


def _knn_body(xyz_ref, idx_ref):
    b = pl.program_id(0)
    rt = pl.program_id(1)
    xs = xyz_ref[0, 0, :].reshape(1, _NP)
    ys = xyz_ref[0, 1, :].reshape(1, _NP)
    zs = xyz_ref[0, 2, :].reshape(1, _NP)
    r0 = rt * _RT
    xr = xyz_ref[0, 0, pl.ds(r0, _RT)].reshape(_RT, 1)
    yr = xyz_ref[0, 1, pl.ds(r0, _RT)].reshape(_RT, 1)
    zr = xyz_ref[0, 2, pl.ds(r0, _RT)].reshape(_RT, 1)
    dx = xr - xs
    dy = yr - ys
    dz = zr - zs
    d2 = (dx * dx + dy * dy) + dz * dz
    ci = lax.broadcasted_iota(jnp.int32, (_RT, _NP), 1)
    off = b * _NPB

    for r in range(_K):
        m = jnp.min(d2, axis=1, keepdims=True)
        sel = jnp.where(d2 == m, ci, _NP)
        j = jnp.min(sel, axis=1, keepdims=True)
        d2 = jnp.where(ci == j, jnp.float32(jnp.inf), d2)
        idx_ref[0, :, r:r + 1] = j + off


def _knn(xyz_pad):
    return pl.pallas_call(
        _knn_body,
        grid=(_B, _NP // _RT),
        in_specs=[pl.BlockSpec((1, 3, _NP), lambda b, rt: (b, 0, 0))],
        out_specs=pl.BlockSpec((1, _RT, _K), lambda b, rt: (b, rt, 0)),
        out_shape=jax.ShapeDtypeStruct((_B, _NP, _K), jnp.int32),
    )(xyz_pad)


def _sc_body(table_hbm, idx_hbm, consts_hbm, out_hbm,
             idx_v, nbr_v, ctr_v, o_v, consts_v, sem):
    wid = lax.axis_index("s") * 2 + lax.axis_index("c")
    pltpu.sync_copy(consts_hbm, consts_v)
    inv_scale = 1.0 / sqrt(float(_K))
    zero = jnp.zeros((16,), jnp.float32)
    neg = jnp.full((16,), -3.0e38, jnp.float32)

    def chunk_body(g, carry):
        c = g * _NW + wid

        @pl.when(c < _NCHUNK)
        def _():
            base_p = c * _CHUNK
            pltpu.sync_copy(idx_hbm.at[pl.ds(c * _CHUNK * _K, _CHUNK * _K)], idx_v)
            pltpu.sync_copy(table_hbm.at[pl.ds(base_p, _CHUNK)], ctr_v)
            cp = pltpu.async_copy(table_hbm.at[idx_v], nbr_v, sem)
            cp.wait()
            for p in range(_CHUNK):
                e0 = p * _K

                def cc_body(cc, _unused):
                    co = cc * 16
                    bcv = consts_v[0, pl.ds(co, 16)]
                    bfv = consts_v[1, pl.ds(co, 16)]
                    gi = ctr_v[p, pl.ds(co, 16)]
                    pi = ctr_v[p, pl.ds(co + _C, 16)]
                    ls = []
                    m = neg
                    for k in range(_K):
                        gj = nbr_v[e0 + k, pl.ds(co, 16)]
                        pj = nbr_v[e0 + k, pl.ds(co + _C, 16)]
                        l = ((pj - pi) + bcv) * ((gj - gi) + bfv) * inv_scale
                        ls.append(l)
                        m = jnp.maximum(m, l)
                    ssum = zero
                    acc = zero
                    for k in range(_K):
                        e = jnp.exp(ls[k] - m)
                        ssum = ssum + e
                        tj = nbr_v[e0 + k, pl.ds(co + 2 * _C, 16)]
                        acc = acc + e * tj
                    o_v[p, pl.ds(co, 16)] = acc / ssum
                    return 0

                lax.fori_loop(0, _C // 16, cc_body, 0)
            pltpu.sync_copy(o_v, out_hbm.at[pl.ds(base_p, _CHUNK)])

        return carry

    lax.fori_loop(0, _GMAX, chunk_body, 0)


def _sc_combine(table, idx_flat, consts):
    mesh = plsc.VectorSubcoreMesh(core_axis_name="c", subcore_axis_name="s")
    f = functools.partial(
        pl.kernel,
        mesh=mesh,
        out_type=jax.ShapeDtypeStruct((_N, _C), jnp.float32),
        scratch_types=[
            pltpu.VMEM((_CHUNK * _K,), jnp.int32),
            pltpu.VMEM((_CHUNK * _K, 3 * _C), jnp.float32),
            pltpu.VMEM((_CHUNK, 3 * _C), jnp.float32),
            pltpu.VMEM((_CHUNK, _C), jnp.float32),
            pltpu.VMEM((2, _C), jnp.float32),
            pltpu.SemaphoreType.DMA,
        ],
    )(_sc_body)
    return f(table, idx_flat, consts)


def _ln_body(u_ref, f_ref, gamma_ref, beta_ref, out_ref):
    x = u_ref[...] + f_ref[...]
    mu = jnp.mean(x, axis=-1, keepdims=True)
    var = jnp.mean(x * x, axis=-1, keepdims=True) - mu * mu
    out_ref[...] = (x - mu) * jax.lax.rsqrt(var + 1e-5) * gamma_ref[...] + beta_ref[...]


def _residual_ln(u, features, gamma, beta):
    rows = 1000
    return pl.pallas_call(
        _ln_body,
        grid=(_N // rows,),
        in_specs=[
            pl.BlockSpec((rows, _C), lambda i: (i, 0)),
            pl.BlockSpec((rows, _C), lambda i: (i, 0)),
            pl.BlockSpec((1, _C), lambda i: (0, 0)),
            pl.BlockSpec((1, _C), lambda i: (0, 0)),
        ],
        out_specs=pl.BlockSpec((rows, _C), lambda i: (i, 0)),
        out_shape=jax.ShapeDtypeStruct((_N, _C), jnp.float32),
    )(u, features, gamma.reshape(1, _C), beta.reshape(1, _C))


def kernel(features, coords, W_ft, b_ft, W_coord, b_coord, W_feat, b_feat, gamma, beta):
    table = _precompute_table(features, coords, W_feat, W_coord, W_ft, b_ft)

    xyz = coords[:, 1:].reshape(_B, _NPB, 3)
    pad = jnp.full((_B, _NP - _NPB, 3), 4.0e8, jnp.float32)
    xyz_pad = jnp.concatenate([xyz, pad], axis=1).transpose(0, 2, 1)
    idx = _knn(xyz_pad)
    idx_flat = idx[:, :_NPB, :].reshape(_N * _K)

    consts = jnp.stack([b_coord, b_feat])
    u = _sc_combine(table, idx_flat, consts)
    return _residual_ln(u, features, gamma, beta)


# speedup vs baseline: 32.8667x; 1.2303x over previous
import functools
from math import sqrt

import jax
import jax.numpy as jnp
from jax import lax
from jax.experimental import pallas as pl
from jax.experimental.pallas import tpu as pltpu
from jax.experimental.pallas import tpu_sc as plsc

_N = 10000
_C = 128
_B = 4
_K = 16
_NPB = _N // _B
_NP = 2560
_RT = 256
_CHUNK = 8
_NCHUNK = _N // _CHUNK
_NW = 32
_GMAX = -(-_NCHUNK // _NW)


def _pre_body(f_ref, c_ref, wfeat_ref, wcoord_ref, wft_ref, bft_ref, out_ref):
    f = f_ref[...]
    xyz = c_ref[:, 1:4]
    out_ref[:, 0:_C] = jnp.dot(f, wfeat_ref[...], preferred_element_type=jnp.float32)
    out_ref[:, _C:2 * _C] = jnp.dot(xyz, wcoord_ref[...], preferred_element_type=jnp.float32)
    out_ref[:, 2 * _C:3 * _C] = (
        jnp.dot(f, wft_ref[...], preferred_element_type=jnp.float32) + bft_ref[...]
    )


def _precompute_table(features, coords, W_feat, W_coord, W_ft, b_ft):
    rows = 1000
    grid = _N // rows
    return pl.pallas_call(
        _pre_body,
        grid=(grid,),
        in_specs=[
            pl.BlockSpec((rows, _C), lambda i: (i, 0)),
            pl.BlockSpec((rows, 4), lambda i: (i, 0)),
            pl.BlockSpec((_C, _C), lambda i: (0, 0)),
            pl.BlockSpec((3, _C), lambda i: (0, 0)),
            pl.BlockSpec((_C, _C), lambda i: (0, 0)),
            pl.BlockSpec((1, _C), lambda i: (0, 0)),
        ],
        out_specs=pl.BlockSpec((rows, 3 * _C), lambda i: (i, 0)),
        out_shape=jax.ShapeDtypeStruct((_N, 3 * _C), jnp.float32),
    )(features, coords, W_feat, W_coord, W_ft, b_ft.reshape(1, _C))


def _knn_body(xyz_ref, idx_ref):
    b = pl.program_id(0)
    rt = pl.program_id(1)
    xs = xyz_ref[0, 0, :].reshape(1, _NP)
    ys = xyz_ref[0, 1, :].reshape(1, _NP)
    zs = xyz_ref[0, 2, :].reshape(1, _NP)
    r0 = rt * _RT
    xr = xyz_ref[0, 0, pl.ds(r0, _RT)].reshape(_RT, 1)
    yr = xyz_ref[0, 1, pl.ds(r0, _RT)].reshape(_RT, 1)
    zr = xyz_ref[0, 2, pl.ds(r0, _RT)].reshape(_RT, 1)
    dx = xr - xs
    dy = yr - ys
    dz = zr - zs
    d2 = (dx * dx + dy * dy) + dz * dz
    cif = lax.broadcasted_iota(jnp.int32, (_RT, _NP), 1).astype(jnp.float32)
    off = b * _NPB

    for r in range(_K):
        m = jnp.min(d2, axis=1, keepdims=True)
        sel = jnp.where(d2 == m, cif, jnp.float32(_NP))
        j = jnp.min(sel, axis=1, keepdims=True)
        d2 = jnp.where(cif == j, jnp.float32(jnp.inf), d2)
        idx_ref[0, :, r:r + 1] = j.astype(jnp.int32) + off


def _knn(xyz_pad):
    return pl.pallas_call(
        _knn_body,
        grid=(_B, _NP // _RT),
        in_specs=[pl.BlockSpec((1, 3, _NP), lambda b, rt: (b, 0, 0))],
        out_specs=pl.BlockSpec((1, _RT, _K), lambda b, rt: (b, rt, 0)),
        out_shape=jax.ShapeDtypeStruct((_B, _NP, _K), jnp.int32),
    )(xyz_pad)


def _sc_body(table_hbm, idx_hbm, consts_hbm, out_hbm,
             idx_v, nbr_v, ctr_v, o_v, consts_v, sems):
    wid = lax.axis_index("s") * 2 + lax.axis_index("c")
    pltpu.sync_copy(consts_hbm, consts_v)
    inv_scale = 1.0 / sqrt(float(_K))
    zero = jnp.zeros((16,), jnp.float32)
    neg = jnp.full((16,), -3.0e38, jnp.float32)

    def prefetch(c, b):
        pltpu.sync_copy(idx_hbm.at[pl.ds(c * _CHUNK * _K, _CHUNK * _K)], idx_v.at[b])
        pltpu.sync_copy(table_hbm.at[pl.ds(c * _CHUNK, _CHUNK)], ctr_v.at[b])
        pltpu.async_copy(table_hbm.at[idx_v.at[b]], nbr_v.at[b], sems[b])

    def process(c, b):
        @pl.when(c < _NCHUNK)
        def _():
            nc = c + _NW

            @pl.when(nc < _NCHUNK)
            def _():
                prefetch(nc, b ^ 1)

            pltpu.make_async_copy(table_hbm.at[idx_v.at[b]], nbr_v.at[b], sems[b]).wait()
            for p in range(_CHUNK):
                e0 = p * _K

                def cc_body(cc, _unused):
                    co = cc * 16
                    bcv = consts_v[0, pl.ds(co, 16)]
                    bfv = consts_v[1, pl.ds(co, 16)]
                    gi = ctr_v[b, p, pl.ds(co, 16)]
                    pi = ctr_v[b, p, pl.ds(co + _C, 16)]
                    ls = []
                    m = neg
                    for k in range(_K):
                        gj = nbr_v[b, e0 + k, pl.ds(co, 16)]
                        pj = nbr_v[b, e0 + k, pl.ds(co + _C, 16)]
                        l = ((pj - pi) + bcv) * ((gj - gi) + bfv) * inv_scale
                        ls.append(l)
                        m = jnp.maximum(m, l)
                    ssum = zero
                    acc = zero
                    for k in range(_K):
                        e = jnp.exp(ls[k] - m)
                        ssum = ssum + e
                        tj = nbr_v[b, e0 + k, pl.ds(co + 2 * _C, 16)]
                        acc = acc + e * tj
                    o_v[p, pl.ds(co, 16)] = acc / ssum
                    return 0

                lax.fori_loop(0, _C // 16, cc_body, 0)
            pltpu.sync_copy(o_v, out_hbm.at[pl.ds(c * _CHUNK, _CHUNK)])

    prefetch(wid, 0)

    def pair_body(gg, carry):
        for sub in range(2):
            process((gg * 2 + sub) * _NW + wid, sub)
        return carry

    lax.fori_loop(0, _GMAX // 2, pair_body, 0)


def _sc_combine(table, idx_flat, consts):
    mesh = plsc.VectorSubcoreMesh(core_axis_name="c", subcore_axis_name="s")
    f = functools.partial(
        pl.kernel,
        mesh=mesh,
        out_type=jax.ShapeDtypeStruct((_N, _C), jnp.float32),
        scratch_types=[
            pltpu.VMEM((2, _CHUNK * _K), jnp.int32),
            pltpu.VMEM((2, _CHUNK * _K, 3 * _C), jnp.float32),
            pltpu.VMEM((2, _CHUNK, 3 * _C), jnp.float32),
            pltpu.VMEM((_CHUNK, _C), jnp.float32),
            pltpu.VMEM((2, _C), jnp.float32),
            [pltpu.SemaphoreType.DMA, pltpu.SemaphoreType.DMA],
        ],
    )(_sc_body)
    return f(table, idx_flat, consts)


def _ln_body(u_ref, f_ref, gamma_ref, beta_ref, out_ref):
    x = u_ref[...] + f_ref[...]
    mu = jnp.mean(x, axis=-1, keepdims=True)
    var = jnp.mean(x * x, axis=-1, keepdims=True) - mu * mu
    out_ref[...] = (x - mu) * jax.lax.rsqrt(var + 1e-5) * gamma_ref[...] + beta_ref[...]


def _residual_ln(u, features, gamma, beta):
    rows = 1000
    return pl.pallas_call(
        _ln_body,
        grid=(_N // rows,),
        in_specs=[
            pl.BlockSpec((rows, _C), lambda i: (i, 0)),
            pl.BlockSpec((rows, _C), lambda i: (i, 0)),
            pl.BlockSpec((1, _C), lambda i: (0, 0)),
            pl.BlockSpec((1, _C), lambda i: (0, 0)),
        ],
        out_specs=pl.BlockSpec((rows, _C), lambda i: (i, 0)),
        out_shape=jax.ShapeDtypeStruct((_N, _C), jnp.float32),
    )(u, features, gamma.reshape(1, _C), beta.reshape(1, _C))


def kernel(features, coords, W_ft, b_ft, W_coord, b_coord, W_feat, b_feat, gamma, beta):
    table = _precompute_table(features, coords, W_feat, W_coord, W_ft, b_ft)

    xyz = coords[:, 1:].reshape(_B, _NPB, 3)
    pad = jnp.full((_B, _NP - _NPB, 3), 4.0e8, jnp.float32)
    xyz_pad = jnp.concatenate([xyz, pad], axis=1).transpose(0, 2, 1)
    idx = _knn(xyz_pad)
    idx_flat = idx[:, :_NPB, :].reshape(_N * _K)

    consts = jnp.stack([b_coord, b_feat])
    u = _sc_combine(table, idx_flat, consts)
    return _residual_ln(u, features, gamma, beta)


# speedup vs baseline: 39.5319x; 1.2028x over previous
import functools
from math import sqrt

import jax
import jax.numpy as jnp
from jax import lax
from jax.experimental import pallas as pl
from jax.experimental.pallas import tpu as pltpu
from jax.experimental.pallas import tpu_sc as plsc

_N = 10000
_C = 128
_B = 4
_K = 16
_NPB = _N // _B
_NP = 2560
_RT = 256
_CHUNK = 8
_NCHUNK = _N // _CHUNK
_NW = 32
_GMAX = -(-_NCHUNK // _NW)


def _pre_body(f_ref, c_ref, wfeat_ref, wcoord_ref, wft_ref, bft_ref, out_ref):
    f = f_ref[...]
    xyz = c_ref[:, 1:4]
    out_ref[:, 0:_C] = jnp.dot(f, wfeat_ref[...], preferred_element_type=jnp.float32)
    out_ref[:, _C:2 * _C] = jnp.dot(xyz, wcoord_ref[...], preferred_element_type=jnp.float32)
    out_ref[:, 2 * _C:3 * _C] = (
        jnp.dot(f, wft_ref[...], preferred_element_type=jnp.float32) + bft_ref[...]
    )


def _precompute_table(features, coords, W_feat, W_coord, W_ft, b_ft):
    rows = 1000
    grid = _N // rows
    return pl.pallas_call(
        _pre_body,
        grid=(grid,),
        in_specs=[
            pl.BlockSpec((rows, _C), lambda i: (i, 0)),
            pl.BlockSpec((rows, 4), lambda i: (i, 0)),
            pl.BlockSpec((_C, _C), lambda i: (0, 0)),
            pl.BlockSpec((3, _C), lambda i: (0, 0)),
            pl.BlockSpec((_C, _C), lambda i: (0, 0)),
            pl.BlockSpec((1, _C), lambda i: (0, 0)),
        ],
        out_specs=pl.BlockSpec((rows, 3 * _C), lambda i: (i, 0)),
        out_shape=jax.ShapeDtypeStruct((_N, 3 * _C), jnp.float32),
    )(features, coords, W_feat, W_coord, W_ft, b_ft.reshape(1, _C))


def _knn_body(xyz_ref, idx_ref, *, base_b):
    b = pl.program_id(0) + base_b
    rt = pl.program_id(1)
    xs = xyz_ref[0, 0, :].reshape(1, _NP)
    ys = xyz_ref[0, 1, :].reshape(1, _NP)
    zs = xyz_ref[0, 2, :].reshape(1, _NP)
    r0 = rt * _RT
    xr = xyz_ref[0, 0, pl.ds(r0, _RT)].reshape(_RT, 1)
    yr = xyz_ref[0, 1, pl.ds(r0, _RT)].reshape(_RT, 1)
    zr = xyz_ref[0, 2, pl.ds(r0, _RT)].reshape(_RT, 1)
    dx = xr - xs
    dy = yr - ys
    dz = zr - zs
    d2 = (dx * dx + dy * dy) + dz * dz
    cif = lax.broadcasted_iota(jnp.int32, (_RT, _NP), 1).astype(jnp.float32)
    off = b * _NPB

    for r in range(_K):
        m = jnp.min(d2, axis=1, keepdims=True)
        sel = jnp.where(d2 == m, cif, jnp.float32(_NP))
        j = jnp.min(sel, axis=1, keepdims=True)
        d2 = jnp.where(cif == j, jnp.float32(jnp.inf), d2)
        idx_ref[0, :, r:r + 1] = j.astype(jnp.int32) + off


def _knn(xyz_pad, base_b, nb):
    body = functools.partial(_knn_body, base_b=base_b)
    return pl.pallas_call(
        body,
        grid=(nb, _NP // _RT),
        in_specs=[pl.BlockSpec((1, 3, _NP), lambda b, rt: (b, 0, 0))],
        out_specs=pl.BlockSpec((1, _RT, _K), lambda b, rt: (b, rt, 0)),
        out_shape=jax.ShapeDtypeStruct((nb, _NP, _K), jnp.int32),
    )(xyz_pad)


def _sc_body(table_hbm, idx_hbm, consts_hbm, out_hbm,
             idx_v, nbr_v, ctr_v, o_v, consts_v, sems, *, p0, nchunk):
    wid = lax.axis_index("s") * 2 + lax.axis_index("c")
    pltpu.sync_copy(consts_hbm, consts_v)
    inv_scale = 1.0 / sqrt(float(_K))
    zero = jnp.zeros((16,), jnp.float32)
    neg = jnp.full((16,), -3.0e38, jnp.float32)
    gmax = -(-nchunk // _NW)
    assert gmax % 2 == 0

    def prefetch(c, b):
        pltpu.sync_copy(idx_hbm.at[pl.ds(c * _CHUNK * _K, _CHUNK * _K)], idx_v.at[b])
        pltpu.sync_copy(table_hbm.at[pl.ds(p0 + c * _CHUNK, _CHUNK)], ctr_v.at[b])
        pltpu.async_copy(table_hbm.at[idx_v.at[b]], nbr_v.at[b], sems[b])

    def process(c, b):
        @pl.when(c < nchunk)
        def _():
            nc = c + _NW

            @pl.when(nc < nchunk)
            def _():
                prefetch(nc, b ^ 1)

            pltpu.make_async_copy(table_hbm.at[idx_v.at[b]], nbr_v.at[b], sems[b]).wait()
            for p in range(_CHUNK):
                e0 = p * _K

                def cc_body(cc, _unused):
                    co = cc * 16
                    bcv = consts_v[0, pl.ds(co, 16)]
                    bfv = consts_v[1, pl.ds(co, 16)]
                    gi = ctr_v[b, p, pl.ds(co, 16)]
                    pi = ctr_v[b, p, pl.ds(co + _C, 16)]
                    ls = []
                    m = neg
                    for k in range(_K):
                        gj = nbr_v[b, e0 + k, pl.ds(co, 16)]
                        pj = nbr_v[b, e0 + k, pl.ds(co + _C, 16)]
                        l = ((pj - pi) + bcv) * ((gj - gi) + bfv) * inv_scale
                        ls.append(l)
                        m = jnp.maximum(m, l)
                    ssum = zero
                    acc = zero
                    for k in range(_K):
                        e = jnp.exp(ls[k] - m)
                        ssum = ssum + e
                        tj = nbr_v[b, e0 + k, pl.ds(co + 2 * _C, 16)]
                        acc = acc + e * tj
                    o_v[p, pl.ds(co, 16)] = acc / ssum
                    return 0

                lax.fori_loop(0, _C // 16, cc_body, 0)
            pltpu.sync_copy(o_v, out_hbm.at[pl.ds(c * _CHUNK, _CHUNK)])

    prefetch(wid, 0)

    def pair_body(gg, carry):
        for sub in range(2):
            process((gg * 2 + sub) * _NW + wid, sub)
        return carry

    lax.fori_loop(0, gmax // 2, pair_body, 0)


def _sc_combine(table, idx_flat, consts, p0, npts):
    mesh = plsc.VectorSubcoreMesh(core_axis_name="c", subcore_axis_name="s")
    body = functools.partial(_sc_body, p0=p0, nchunk=npts // _CHUNK)
    f = functools.partial(
        pl.kernel,
        mesh=mesh,
        out_type=jax.ShapeDtypeStruct((npts, _C), jnp.float32),
        scratch_types=[
            pltpu.VMEM((2, _CHUNK * _K), jnp.int32),
            pltpu.VMEM((2, _CHUNK * _K, 3 * _C), jnp.float32),
            pltpu.VMEM((2, _CHUNK, 3 * _C), jnp.float32),
            pltpu.VMEM((_CHUNK, _C), jnp.float32),
            pltpu.VMEM((2, _C), jnp.float32),
            [pltpu.SemaphoreType.DMA, pltpu.SemaphoreType.DMA],
        ],
    )(body)
    return f(table, idx_flat, consts)


def _ln_body(u_ref, f_ref, gamma_ref, beta_ref, out_ref):
    x = u_ref[...] + f_ref[...]
    mu = jnp.mean(x, axis=-1, keepdims=True)
    var = jnp.mean(x * x, axis=-1, keepdims=True) - mu * mu
    out_ref[...] = (x - mu) * jax.lax.rsqrt(var + 1e-5) * gamma_ref[...] + beta_ref[...]


def _residual_ln(u, features, gamma, beta):
    rows = 1000
    return pl.pallas_call(
        _ln_body,
        grid=(_N // rows,),
        in_specs=[
            pl.BlockSpec((rows, _C), lambda i: (i, 0)),
            pl.BlockSpec((rows, _C), lambda i: (i, 0)),
            pl.BlockSpec((1, _C), lambda i: (0, 0)),
            pl.BlockSpec((1, _C), lambda i: (0, 0)),
        ],
        out_specs=pl.BlockSpec((rows, _C), lambda i: (i, 0)),
        out_shape=jax.ShapeDtypeStruct((_N, _C), jnp.float32),
    )(u, features, gamma.reshape(1, _C), beta.reshape(1, _C))


def kernel(features, coords, W_ft, b_ft, W_coord, b_coord, W_feat, b_feat, gamma, beta):
    table = _precompute_table(features, coords, W_feat, W_coord, W_ft, b_ft)

    xyz = coords[:, 1:].reshape(_B, _NPB, 3)
    pad = jnp.full((_B, _NP - _NPB, 3), 4.0e8, jnp.float32)
    xyz_pad = jnp.concatenate([xyz, pad], axis=1).transpose(0, 2, 1)
    consts = jnp.stack([b_coord, b_feat])

    halves = []
    nbh = _B // 2
    for h in range(2):
        idx_h = _knn(xyz_pad[h * nbh:(h + 1) * nbh], h * nbh, nbh)
        idx_flat = idx_h[:, :_NPB, :].reshape(nbh * _NPB * _K)
        halves.append(_sc_combine(table, idx_flat, consts, h * nbh * _NPB, nbh * _NPB))

    u = jnp.concatenate(halves, axis=0)
    return _residual_ln(u, features, gamma, beta)


# speedup vs baseline: 40.4946x; 1.0244x over previous
import functools
from math import sqrt

import jax
import jax.numpy as jnp
from jax import lax
from jax.experimental import pallas as pl
from jax.experimental.pallas import tpu as pltpu
from jax.experimental.pallas import tpu_sc as plsc

_N = 10000
_C = 128
_B = 4
_K = 16
_NPB = _N // _B
_NP = 2560
_RT = 256
_CHUNK = 8
_NCHUNK = _N // _CHUNK
_NW = 32
_GMAX = -(-_NCHUNK // _NW)


def _pre_body(f_ref, c_ref, wfeat_ref, wcoord_ref, wft_ref, bft_ref, out_ref):
    f = f_ref[...]
    xyz = c_ref[:, 1:4]
    out_ref[:, 0:_C] = jnp.dot(f, wfeat_ref[...], preferred_element_type=jnp.float32)
    out_ref[:, _C:2 * _C] = jnp.dot(xyz, wcoord_ref[...], preferred_element_type=jnp.float32)
    out_ref[:, 2 * _C:3 * _C] = (
        jnp.dot(f, wft_ref[...], preferred_element_type=jnp.float32) + bft_ref[...]
    )


def _precompute_table(features, coords, W_feat, W_coord, W_ft, b_ft):
    rows = 1000
    grid = _N // rows
    return pl.pallas_call(
        _pre_body,
        grid=(grid,),
        in_specs=[
            pl.BlockSpec((rows, _C), lambda i: (i, 0)),
            pl.BlockSpec((rows, 4), lambda i: (i, 0)),
            pl.BlockSpec((_C, _C), lambda i: (0, 0)),
            pl.BlockSpec((3, _C), lambda i: (0, 0)),
            pl.BlockSpec((_C, _C), lambda i: (0, 0)),
            pl.BlockSpec((1, _C), lambda i: (0, 0)),
        ],
        out_specs=pl.BlockSpec((rows, 3 * _C), lambda i: (i, 0)),
        out_shape=jax.ShapeDtypeStruct((_N, 3 * _C), jnp.float32),
    )(features, coords, W_feat, W_coord, W_ft, b_ft.reshape(1, _C))


def _knn_body(xyz_ref, idx_ref, *, base_b):
    b = pl.program_id(0) + base_b
    rt = pl.program_id(1)
    xs = xyz_ref[0, 0, :].reshape(1, _NP)
    ys = xyz_ref[0, 1, :].reshape(1, _NP)
    zs = xyz_ref[0, 2, :].reshape(1, _NP)
    r0 = rt * _RT
    xr = xyz_ref[0, 0, pl.ds(r0, _RT)].reshape(_RT, 1)
    yr = xyz_ref[0, 1, pl.ds(r0, _RT)].reshape(_RT, 1)
    zr = xyz_ref[0, 2, pl.ds(r0, _RT)].reshape(_RT, 1)
    dx = xr - xs
    dy = yr - ys
    dz = zr - zs
    d2 = (dx * dx + dy * dy) + dz * dz
    cif = lax.broadcasted_iota(jnp.int32, (_RT, _NP), 1).astype(jnp.float32)
    off = b * _NPB

    for r in range(_K):
        m = jnp.min(d2, axis=1, keepdims=True)
        sel = jnp.where(d2 == m, cif, jnp.float32(_NP))
        j = jnp.min(sel, axis=1, keepdims=True)
        d2 = jnp.where(cif == j, jnp.float32(jnp.inf), d2)
        idx_ref[0, :, r:r + 1] = j.astype(jnp.int32) + off


def _knn(xyz_pad, base_b, nb):
    body = functools.partial(_knn_body, base_b=base_b)
    return pl.pallas_call(
        body,
        grid=(nb, _NP // _RT),
        in_specs=[pl.BlockSpec((1, 3, _NP), lambda b, rt: (b, 0, 0))],
        out_specs=pl.BlockSpec((1, _RT, _K), lambda b, rt: (b, rt, 0)),
        out_shape=jax.ShapeDtypeStruct((nb, _NP, _K), jnp.int32),
    )(xyz_pad)


def _sc_body(table_hbm, idx_hbm, consts_hbm, out_hbm,
             idx_v, nbr_v, ctr_v, o_v, consts_v, sems, *, p0, nchunk):
    wid = lax.axis_index("s") * 2 + lax.axis_index("c")
    pltpu.sync_copy(consts_hbm, consts_v)
    inv_scale = 1.0 / sqrt(float(_K))
    zero = jnp.zeros((16,), jnp.float32)
    neg = jnp.full((16,), -3.0e38, jnp.float32)
    gmax = -(-nchunk // _NW)
    assert gmax % 2 == 0

    def prefetch(c, b):
        pltpu.sync_copy(idx_hbm.at[pl.ds(c * _CHUNK * _K, _CHUNK * _K)], idx_v.at[b])
        pltpu.async_copy(table_hbm.at[pl.ds(p0 + c * _CHUNK, _CHUNK)], ctr_v.at[b], sems[2 + b])
        pltpu.async_copy(table_hbm.at[idx_v.at[b]], nbr_v.at[b], sems[b])

    def process(c, b):
        @pl.when(c < nchunk)
        def _():
            nc = c + _NW

            @pl.when(nc < nchunk)
            def _():
                prefetch(nc, b ^ 1)

            pltpu.make_async_copy(table_hbm.at[pl.ds(p0 + c * _CHUNK, _CHUNK)], ctr_v.at[b], sems[2 + b]).wait()
            pltpu.make_async_copy(table_hbm.at[idx_v.at[b]], nbr_v.at[b], sems[b]).wait()
            for p in range(_CHUNK):
                e0 = p * _K

                def cc_body(cc, _unused):
                    co = cc * 16
                    bcv = consts_v[0, pl.ds(co, 16)]
                    bfv = consts_v[1, pl.ds(co, 16)]
                    gi = ctr_v[b, p, pl.ds(co, 16)]
                    pi = ctr_v[b, p, pl.ds(co + _C, 16)]
                    ls = []
                    m = neg
                    for k in range(_K):
                        gj = nbr_v[b, e0 + k, pl.ds(co, 16)]
                        pj = nbr_v[b, e0 + k, pl.ds(co + _C, 16)]
                        l = ((pj - pi) + bcv) * ((gj - gi) + bfv) * inv_scale
                        ls.append(l)
                        m = jnp.maximum(m, l)
                    ssum = zero
                    acc = zero
                    for k in range(_K):
                        e = jnp.exp(ls[k] - m)
                        ssum = ssum + e
                        tj = nbr_v[b, e0 + k, pl.ds(co + 2 * _C, 16)]
                        acc = acc + e * tj
                    o_v[p, pl.ds(co, 16)] = acc / ssum
                    return 0

                lax.fori_loop(0, _C // 16, cc_body, 0)
            pltpu.sync_copy(o_v, out_hbm.at[pl.ds(c * _CHUNK, _CHUNK)])

    prefetch(wid, 0)

    def pair_body(gg, carry):
        for sub in range(2):
            process((gg * 2 + sub) * _NW + wid, sub)
        return carry

    lax.fori_loop(0, gmax // 2, pair_body, 0)


def _sc_combine(table, idx_flat, consts, p0, npts):
    mesh = plsc.VectorSubcoreMesh(core_axis_name="c", subcore_axis_name="s")
    body = functools.partial(_sc_body, p0=p0, nchunk=npts // _CHUNK)
    f = functools.partial(
        pl.kernel,
        mesh=mesh,
        out_type=jax.ShapeDtypeStruct((npts, _C), jnp.float32),
        scratch_types=[
            pltpu.VMEM((2, _CHUNK * _K), jnp.int32),
            pltpu.VMEM((2, _CHUNK * _K, 3 * _C), jnp.float32),
            pltpu.VMEM((2, _CHUNK, 3 * _C), jnp.float32),
            pltpu.VMEM((_CHUNK, _C), jnp.float32),
            pltpu.VMEM((2, _C), jnp.float32),
            [pltpu.SemaphoreType.DMA, pltpu.SemaphoreType.DMA,
             pltpu.SemaphoreType.DMA, pltpu.SemaphoreType.DMA],
        ],
    )(body)
    return f(table, idx_flat, consts)


def _ln_body(u_ref, f_ref, gamma_ref, beta_ref, out_ref):
    x = u_ref[...] + f_ref[...]
    mu = jnp.mean(x, axis=-1, keepdims=True)
    var = jnp.mean(x * x, axis=-1, keepdims=True) - mu * mu
    out_ref[...] = (x - mu) * jax.lax.rsqrt(var + 1e-5) * gamma_ref[...] + beta_ref[...]


def _residual_ln(u, features, gamma, beta):
    rows = 1000
    return pl.pallas_call(
        _ln_body,
        grid=(_N // rows,),
        in_specs=[
            pl.BlockSpec((rows, _C), lambda i: (i, 0)),
            pl.BlockSpec((rows, _C), lambda i: (i, 0)),
            pl.BlockSpec((1, _C), lambda i: (0, 0)),
            pl.BlockSpec((1, _C), lambda i: (0, 0)),
        ],
        out_specs=pl.BlockSpec((rows, _C), lambda i: (i, 0)),
        out_shape=jax.ShapeDtypeStruct((_N, _C), jnp.float32),
    )(u, features, gamma.reshape(1, _C), beta.reshape(1, _C))


def kernel(features, coords, W_ft, b_ft, W_coord, b_coord, W_feat, b_feat, gamma, beta):
    table = _precompute_table(features, coords, W_feat, W_coord, W_ft, b_ft)

    xyz = coords[:, 1:].reshape(_B, _NPB, 3)
    pad = jnp.full((_B, _NP - _NPB, 3), 4.0e8, jnp.float32)
    xyz_pad = jnp.concatenate([xyz, pad], axis=1).transpose(0, 2, 1)
    consts = jnp.stack([b_coord, b_feat])

    halves = []
    nbh = _B // 2
    for h in range(2):
        idx_h = _knn(xyz_pad[h * nbh:(h + 1) * nbh], h * nbh, nbh)
        idx_flat = idx_h[:, :_NPB, :].reshape(nbh * _NPB * _K)
        halves.append(_sc_combine(table, idx_flat, consts, h * nbh * _NPB, nbh * _NPB))

    u = jnp.concatenate(halves, axis=0)
    return _residual_ln(u, features, gamma, beta)


# speedup vs baseline: 41.0027x; 1.0125x over previous
import functools
from math import sqrt

import jax
import jax.numpy as jnp
from jax import lax
from jax.experimental import pallas as pl
from jax.experimental.pallas import tpu as pltpu
from jax.experimental.pallas import tpu_sc as plsc

_N = 10000
_C = 128
_B = 4
_K = 16
_NPB = _N // _B
_NP = 2560
_RT = 256
_CHUNK = 8
_NCHUNK = _N // _CHUNK
_NW = 32
_GMAX = -(-_NCHUNK // _NW)


def _pre_body(f_ref, c_ref, wfeat_ref, wcoord_ref, wft_ref, bft_ref, out_ref):
    f = f_ref[...]
    xyz = c_ref[:, 1:4]
    out_ref[:, 0:_C] = jnp.dot(f, wfeat_ref[...], preferred_element_type=jnp.float32)
    out_ref[:, _C:2 * _C] = jnp.dot(xyz, wcoord_ref[...], preferred_element_type=jnp.float32)
    out_ref[:, 2 * _C:3 * _C] = (
        jnp.dot(f, wft_ref[...], preferred_element_type=jnp.float32) + bft_ref[...]
    )


def _precompute_table(features, coords, W_feat, W_coord, W_ft, b_ft):
    rows = 1000
    grid = _N // rows
    return pl.pallas_call(
        _pre_body,
        grid=(grid,),
        in_specs=[
            pl.BlockSpec((rows, _C), lambda i: (i, 0)),
            pl.BlockSpec((rows, 4), lambda i: (i, 0)),
            pl.BlockSpec((_C, _C), lambda i: (0, 0)),
            pl.BlockSpec((3, _C), lambda i: (0, 0)),
            pl.BlockSpec((_C, _C), lambda i: (0, 0)),
            pl.BlockSpec((1, _C), lambda i: (0, 0)),
        ],
        out_specs=pl.BlockSpec((rows, 3 * _C), lambda i: (i, 0)),
        out_shape=jax.ShapeDtypeStruct((_N, 3 * _C), jnp.float32),
    )(features, coords, W_feat, W_coord, W_ft, b_ft.reshape(1, _C))


def _knn_body(xyz_ref, idx_ref, *, base_b):
    b = pl.program_id(0) + base_b
    rt = pl.program_id(1)
    xs = xyz_ref[0, 0, :].reshape(1, _NP)
    ys = xyz_ref[0, 1, :].reshape(1, _NP)
    zs = xyz_ref[0, 2, :].reshape(1, _NP)
    r0 = rt * _RT
    xr = xyz_ref[0, 0, pl.ds(r0, _RT)].reshape(_RT, 1)
    yr = xyz_ref[0, 1, pl.ds(r0, _RT)].reshape(_RT, 1)
    zr = xyz_ref[0, 2, pl.ds(r0, _RT)].reshape(_RT, 1)
    dx = xr - xs
    dy = yr - ys
    dz = zr - zs
    d2 = (dx * dx + dy * dy) + dz * dz
    cif = lax.broadcasted_iota(jnp.int32, (_RT, _NP), 1).astype(jnp.float32)
    off = b * _NPB

    rowi = lax.broadcasted_iota(jnp.int32, (_RT, 1), 0) + r0
    d2 = jnp.where(cif == rowi.astype(jnp.float32), jnp.float32(jnp.inf), d2)
    idx_ref[0, :, 0:1] = rowi + off

    for r in range(1, _K):
        m = jnp.min(d2, axis=1, keepdims=True)
        sel = jnp.where(d2 == m, cif, jnp.float32(_NP))
        j = jnp.min(sel, axis=1, keepdims=True)
        d2 = jnp.where(cif == j, jnp.float32(jnp.inf), d2)
        idx_ref[0, :, r:r + 1] = j.astype(jnp.int32) + off


def _knn(xyz_pad, base_b, nb):
    body = functools.partial(_knn_body, base_b=base_b)
    return pl.pallas_call(
        body,
        grid=(nb, _NP // _RT),
        in_specs=[pl.BlockSpec((1, 3, _NP), lambda b, rt: (b, 0, 0))],
        out_specs=pl.BlockSpec((1, _RT, _K), lambda b, rt: (b, rt, 0)),
        out_shape=jax.ShapeDtypeStruct((nb, _NP, _K), jnp.int32),
    )(xyz_pad)


def _sc_body(table_hbm, idx_hbm, consts_hbm, out_hbm,
             idx_v, nbr_v, ctr_v, o_v, consts_v, sems, *, p0, nchunk):
    wid = lax.axis_index("s") * 2 + lax.axis_index("c")
    pltpu.sync_copy(consts_hbm, consts_v)
    inv_scale = 1.0 / sqrt(float(_K))
    zero = jnp.zeros((16,), jnp.float32)
    neg = jnp.full((16,), -3.0e38, jnp.float32)
    gmax = -(-nchunk // _NW)
    assert gmax % 2 == 0

    def prefetch(c, b):
        pltpu.sync_copy(idx_hbm.at[pl.ds(c * _CHUNK * _K, _CHUNK * _K)], idx_v.at[b])
        pltpu.async_copy(table_hbm.at[pl.ds(p0 + c * _CHUNK, _CHUNK)], ctr_v.at[b], sems[2 + b])
        pltpu.async_copy(table_hbm.at[idx_v.at[b]], nbr_v.at[b], sems[b])

    def process(c, b):
        @pl.when(c < nchunk)
        def _():
            nc = c + _NW

            @pl.when(nc < nchunk)
            def _():
                prefetch(nc, b ^ 1)

            pltpu.make_async_copy(table_hbm.at[pl.ds(p0 + c * _CHUNK, _CHUNK)], ctr_v.at[b], sems[2 + b]).wait()
            pltpu.make_async_copy(table_hbm.at[idx_v.at[b]], nbr_v.at[b], sems[b]).wait()
            for p in range(_CHUNK):
                e0 = p * _K

                def cc_body(cc, _unused):
                    co = cc * 16
                    bcv = consts_v[0, pl.ds(co, 16)]
                    bfv = consts_v[1, pl.ds(co, 16)]
                    gi = ctr_v[b, p, pl.ds(co, 16)]
                    pi = ctr_v[b, p, pl.ds(co + _C, 16)]
                    ls = []
                    m = neg
                    for k in range(_K):
                        gj = nbr_v[b, e0 + k, pl.ds(co, 16)]
                        pj = nbr_v[b, e0 + k, pl.ds(co + _C, 16)]
                        l = ((pj - pi) + bcv) * ((gj - gi) + bfv) * inv_scale
                        ls.append(l)
                        m = jnp.maximum(m, l)
                    ssum = zero
                    acc = zero
                    for k in range(_K):
                        e = jnp.exp(ls[k] - m)
                        ssum = ssum + e
                        tj = nbr_v[b, e0 + k, pl.ds(co + 2 * _C, 16)]
                        acc = acc + e * tj
                    o_v[p, pl.ds(co, 16)] = acc / ssum
                    return 0

                lax.fori_loop(0, _C // 16, cc_body, 0)
            pltpu.sync_copy(o_v, out_hbm.at[pl.ds(c * _CHUNK, _CHUNK)])

    prefetch(wid, 0)

    def pair_body(gg, carry):
        for sub in range(2):
            process((gg * 2 + sub) * _NW + wid, sub)
        return carry

    lax.fori_loop(0, gmax // 2, pair_body, 0)


def _sc_combine(table, idx_flat, consts, p0, npts):
    mesh = plsc.VectorSubcoreMesh(core_axis_name="c", subcore_axis_name="s")
    body = functools.partial(_sc_body, p0=p0, nchunk=npts // _CHUNK)
    f = functools.partial(
        pl.kernel,
        mesh=mesh,
        out_type=jax.ShapeDtypeStruct((npts, _C), jnp.float32),
        scratch_types=[
            pltpu.VMEM((2, _CHUNK * _K), jnp.int32),
            pltpu.VMEM((2, _CHUNK * _K, 3 * _C), jnp.float32),
            pltpu.VMEM((2, _CHUNK, 3 * _C), jnp.float32),
            pltpu.VMEM((_CHUNK, _C), jnp.float32),
            pltpu.VMEM((2, _C), jnp.float32),
            [pltpu.SemaphoreType.DMA, pltpu.SemaphoreType.DMA,
             pltpu.SemaphoreType.DMA, pltpu.SemaphoreType.DMA],
        ],
    )(body)
    return f(table, idx_flat, consts)


def _ln_body(u_ref, f_ref, gamma_ref, beta_ref, out_ref):
    x = u_ref[...] + f_ref[...]
    mu = jnp.mean(x, axis=-1, keepdims=True)
    var = jnp.mean(x * x, axis=-1, keepdims=True) - mu * mu
    out_ref[...] = (x - mu) * jax.lax.rsqrt(var + 1e-5) * gamma_ref[...] + beta_ref[...]


def _residual_ln(u, features, gamma, beta):
    rows = 1000
    return pl.pallas_call(
        _ln_body,
        grid=(_N // rows,),
        in_specs=[
            pl.BlockSpec((rows, _C), lambda i: (i, 0)),
            pl.BlockSpec((rows, _C), lambda i: (i, 0)),
            pl.BlockSpec((1, _C), lambda i: (0, 0)),
            pl.BlockSpec((1, _C), lambda i: (0, 0)),
        ],
        out_specs=pl.BlockSpec((rows, _C), lambda i: (i, 0)),
        out_shape=jax.ShapeDtypeStruct((_N, _C), jnp.float32),
    )(u, features, gamma.reshape(1, _C), beta.reshape(1, _C))


def kernel(features, coords, W_ft, b_ft, W_coord, b_coord, W_feat, b_feat, gamma, beta):
    table = _precompute_table(features, coords, W_feat, W_coord, W_ft, b_ft)

    xyz = coords[:, 1:].reshape(_B, _NPB, 3)
    pad = jnp.full((_B, _NP - _NPB, 3), 4.0e8, jnp.float32)
    xyz_pad = jnp.concatenate([xyz, pad], axis=1).transpose(0, 2, 1)
    consts = jnp.stack([b_coord, b_feat])

    halves = []
    nbh = _B // 2
    for h in range(2):
        idx_h = _knn(xyz_pad[h * nbh:(h + 1) * nbh], h * nbh, nbh)
        idx_flat = idx_h[:, :_NPB, :].reshape(nbh * _NPB * _K)
        halves.append(_sc_combine(table, idx_flat, consts, h * nbh * _NPB, nbh * _NPB))

    u = jnp.concatenate(halves, axis=0)
    return _residual_ln(u, features, gamma, beta)


# speedup vs baseline: 42.0761x; 1.0262x over previous
import functools
from math import sqrt

import jax
import jax.numpy as jnp
from jax import lax
from jax.experimental import pallas as pl
from jax.experimental.pallas import tpu as pltpu
from jax.experimental.pallas import tpu_sc as plsc

_N = 10000
_C = 128
_B = 4
_K = 16
_NPB = _N // _B
_NP = 2560
_RT = 256
_CHUNK = 8
_NCHUNK = _N // _CHUNK
_NW = 32
_GMAX = -(-_NCHUNK // _NW)


def _knn_body(xyz_ref, idx_ref, *, base_b):
    b = pl.program_id(0) + base_b
    rt = pl.program_id(1)
    xs = xyz_ref[0, 0, :].reshape(1, _NP)
    ys = xyz_ref[0, 1, :].reshape(1, _NP)
    zs = xyz_ref[0, 2, :].reshape(1, _NP)
    r0 = rt * _RT
    xr = xyz_ref[0, 0, pl.ds(r0, _RT)].reshape(_RT, 1)
    yr = xyz_ref[0, 1, pl.ds(r0, _RT)].reshape(_RT, 1)
    zr = xyz_ref[0, 2, pl.ds(r0, _RT)].reshape(_RT, 1)
    dx = xr - xs
    dy = yr - ys
    dz = zr - zs
    d2 = (dx * dx + dy * dy) + dz * dz
    cif = lax.broadcasted_iota(jnp.int32, (_RT, _NP), 1).astype(jnp.float32)
    off = b * _NPB

    rowi = lax.broadcasted_iota(jnp.int32, (_RT, 1), 0) + r0
    d2 = jnp.where(cif == rowi.astype(jnp.float32), jnp.float32(jnp.inf), d2)
    idx_ref[0, :, 0:1] = rowi + off

    for r in range(1, _K):
        m = jnp.min(d2, axis=1, keepdims=True)
        sel = jnp.where(d2 == m, cif, jnp.float32(_NP))
        j = jnp.min(sel, axis=1, keepdims=True)
        d2 = jnp.where(cif == j, jnp.float32(jnp.inf), d2)
        idx_ref[0, :, r:r + 1] = j.astype(jnp.int32) + off


def _knn(xyz_pad, base_b, nb):
    body = functools.partial(_knn_body, base_b=base_b)
    return pl.pallas_call(
        body,
        grid=(nb, _NP // _RT),
        in_specs=[pl.BlockSpec((1, 3, _NP), lambda b, rt: (b, 0, 0))],
        out_specs=pl.BlockSpec((1, _RT, _K), lambda b, rt: (b, rt, 0)),
        out_shape=jax.ShapeDtypeStruct((nb, _NP, _K), jnp.int32),
    )(xyz_pad)


_TROWS = 512
_NPAD = _TROWS * 20


def _knn_pre_body(xyz_ref, f_ref, c_ref, wfeat_ref, wcoord_ref, wft_ref,
                  bft_ref, idx_ref, table_ref):
    f = f_ref[...]
    xyz3 = c_ref[:, 1:4]
    table_ref[:, 0:_C] = jnp.dot(f, wfeat_ref[...], preferred_element_type=jnp.float32)
    table_ref[:, _C:2 * _C] = jnp.dot(xyz3, wcoord_ref[...], preferred_element_type=jnp.float32)
    table_ref[:, 2 * _C:3 * _C] = (
        jnp.dot(f, wft_ref[...], preferred_element_type=jnp.float32) + bft_ref[...]
    )
    _knn_body(xyz_ref, idx_ref, base_b=0)


def _knn_with_table(xyz_pad, nb, features, coords, W_feat, W_coord, W_ft, b_ft):
    return pl.pallas_call(
        _knn_pre_body,
        grid=(nb, _NP // _RT),
        in_specs=[
            pl.BlockSpec((1, 3, _NP), lambda b, rt: (b, 0, 0)),
            pl.BlockSpec((_TROWS, _C), lambda b, rt: (b * 10 + rt, 0)),
            pl.BlockSpec((_TROWS, 4), lambda b, rt: (b * 10 + rt, 0)),
            pl.BlockSpec((_C, _C), lambda b, rt: (0, 0)),
            pl.BlockSpec((3, _C), lambda b, rt: (0, 0)),
            pl.BlockSpec((_C, _C), lambda b, rt: (0, 0)),
            pl.BlockSpec((1, _C), lambda b, rt: (0, 0)),
        ],
        out_specs=[
            pl.BlockSpec((1, _RT, _K), lambda b, rt: (b, rt, 0)),
            pl.BlockSpec((_TROWS, 3 * _C), lambda b, rt: (b * 10 + rt, 0)),
        ],
        out_shape=[
            jax.ShapeDtypeStruct((nb, _NP, _K), jnp.int32),
            jax.ShapeDtypeStruct((_NPAD, 3 * _C), jnp.float32),
        ],
    )(xyz_pad, features, coords, W_feat, W_coord, W_ft, b_ft.reshape(1, _C))


def _sc_body(table_hbm, idx_hbm, consts_hbm, out_hbm,
             idx_v, nbr_v, ctr_v, o_v, consts_v, sems, *, p0, nchunk):
    wid = lax.axis_index("s") * 2 + lax.axis_index("c")
    pltpu.sync_copy(consts_hbm, consts_v)
    inv_scale = 1.0 / sqrt(float(_K))
    zero = jnp.zeros((16,), jnp.float32)
    neg = jnp.full((16,), -3.0e38, jnp.float32)
    gmax = -(-nchunk // _NW)
    assert gmax % 2 == 0

    def prefetch(c, b):
        pltpu.sync_copy(idx_hbm.at[pl.ds(c * _CHUNK * _K, _CHUNK * _K)], idx_v.at[b])
        pltpu.async_copy(table_hbm.at[pl.ds(p0 + c * _CHUNK, _CHUNK)], ctr_v.at[b], sems[2 + b])
        pltpu.async_copy(table_hbm.at[idx_v.at[b]], nbr_v.at[b], sems[b])

    def process(c, b):
        @pl.when(c < nchunk)
        def _():
            nc = c + _NW

            @pl.when(nc < nchunk)
            def _():
                prefetch(nc, b ^ 1)

            pltpu.make_async_copy(table_hbm.at[pl.ds(p0 + c * _CHUNK, _CHUNK)], ctr_v.at[b], sems[2 + b]).wait()
            pltpu.make_async_copy(table_hbm.at[idx_v.at[b]], nbr_v.at[b], sems[b]).wait()
            for p in range(_CHUNK):
                e0 = p * _K

                def cc_body(cc, _unused):
                    co = cc * 16
                    bcv = consts_v[0, pl.ds(co, 16)]
                    bfv = consts_v[1, pl.ds(co, 16)]
                    gi = ctr_v[b, p, pl.ds(co, 16)]
                    pi = ctr_v[b, p, pl.ds(co + _C, 16)]
                    ls = []
                    m = neg
                    for k in range(_K):
                        gj = nbr_v[b, e0 + k, pl.ds(co, 16)]
                        pj = nbr_v[b, e0 + k, pl.ds(co + _C, 16)]
                        l = ((pj - pi) + bcv) * ((gj - gi) + bfv) * inv_scale
                        ls.append(l)
                        m = jnp.maximum(m, l)
                    ssum = zero
                    acc = zero
                    for k in range(_K):
                        e = jnp.exp(ls[k] - m)
                        ssum = ssum + e
                        tj = nbr_v[b, e0 + k, pl.ds(co + 2 * _C, 16)]
                        acc = acc + e * tj
                    o_v[p, pl.ds(co, 16)] = acc / ssum
                    return 0

                lax.fori_loop(0, _C // 16, cc_body, 0)
            pltpu.sync_copy(o_v, out_hbm.at[pl.ds(c * _CHUNK, _CHUNK)])

    prefetch(wid, 0)

    def pair_body(gg, carry):
        for sub in range(2):
            process((gg * 2 + sub) * _NW + wid, sub)
        return carry

    lax.fori_loop(0, gmax // 2, pair_body, 0)


def _sc_combine(table, idx_flat, consts, p0, npts):
    mesh = plsc.VectorSubcoreMesh(core_axis_name="c", subcore_axis_name="s")
    body = functools.partial(_sc_body, p0=p0, nchunk=npts // _CHUNK)
    f = functools.partial(
        pl.kernel,
        mesh=mesh,
        out_type=jax.ShapeDtypeStruct((npts, _C), jnp.float32),
        scratch_types=[
            pltpu.VMEM((2, _CHUNK * _K), jnp.int32),
            pltpu.VMEM((2, _CHUNK * _K, 3 * _C), jnp.float32),
            pltpu.VMEM((2, _CHUNK, 3 * _C), jnp.float32),
            pltpu.VMEM((_CHUNK, _C), jnp.float32),
            pltpu.VMEM((2, _C), jnp.float32),
            [pltpu.SemaphoreType.DMA, pltpu.SemaphoreType.DMA,
             pltpu.SemaphoreType.DMA, pltpu.SemaphoreType.DMA],
        ],
    )(body)
    return f(table, idx_flat, consts)


def _ln_body(u_ref, f_ref, gamma_ref, beta_ref, out_ref):
    x = u_ref[...] + f_ref[...]
    mu = jnp.mean(x, axis=-1, keepdims=True)
    var = jnp.mean(x * x, axis=-1, keepdims=True) - mu * mu
    out_ref[...] = (x - mu) * jax.lax.rsqrt(var + 1e-5) * gamma_ref[...] + beta_ref[...]


def _residual_ln(u, features, gamma, beta):
    rows = 1000
    return pl.pallas_call(
        _ln_body,
        grid=(_N // rows,),
        in_specs=[
            pl.BlockSpec((rows, _C), lambda i: (i, 0)),
            pl.BlockSpec((rows, _C), lambda i: (i, 0)),
            pl.BlockSpec((1, _C), lambda i: (0, 0)),
            pl.BlockSpec((1, _C), lambda i: (0, 0)),
        ],
        out_specs=pl.BlockSpec((rows, _C), lambda i: (i, 0)),
        out_shape=jax.ShapeDtypeStruct((_N, _C), jnp.float32),
    )(u, features, gamma.reshape(1, _C), beta.reshape(1, _C))


def kernel(features, coords, W_ft, b_ft, W_coord, b_coord, W_feat, b_feat, gamma, beta):
    xyz = coords[:, 1:].reshape(_B, _NPB, 3)
    pad = jnp.full((_B, _NP - _NPB, 3), 4.0e8, jnp.float32)
    xyz_pad = jnp.concatenate([xyz, pad], axis=1).transpose(0, 2, 1)
    consts = jnp.stack([b_coord, b_feat])

    nbh = _B // 2
    idx_0, table = _knn_with_table(
        xyz_pad[:nbh], nbh, features, coords, W_feat, W_coord, W_ft, b_ft)
    halves = []
    for h in range(2):
        if h == 0:
            idx_h = idx_0
        else:
            idx_h = _knn(xyz_pad[h * nbh:(h + 1) * nbh], h * nbh, nbh)
        idx_flat = idx_h[:, :_NPB, :].reshape(nbh * _NPB * _K)
        halves.append(_sc_combine(table, idx_flat, consts, h * nbh * _NPB, nbh * _NPB))

    u = jnp.concatenate(halves, axis=0)
    return _residual_ln(u, features, gamma, beta)
